# Initial kernel scaffold; baseline (speedup 1.0000x reference)
import jax
import jax.numpy as jnp
from jax.experimental import pallas as pl


def kernel(target, context, negatives, emb, ctx_emb):
    raise NotImplementedError("write your pallas kernel here")


# speedup vs baseline: 4.0941x; 4.0941x over previous
import functools

import jax
import jax.numpy as jnp
from jax import lax
from jax.experimental import pallas as pl
from jax.experimental.pallas import tpu as pltpu
from jax.experimental.pallas import tpu_sc as plsc

NC = 2
NS = 16
NW = NC * NS
LANES = 16


@functools.lru_cache(maxsize=None)
def _make_sc_scores(B, K, D):
    KP1 = K + 1
    EPW = B // NW
    C = 32
    NCHUNK = EPW // C
    NG = C // LANES
    NSPLIT = 4
    SEG = C * K // NSPLIT

    mesh = plsc.VectorSubcoreMesh(core_axis_name="c", subcore_axis_name="s")

    @functools.partial(
        pl.kernel,
        mesh=mesh,
        compiler_params=pltpu.CompilerParams(needs_layout_passes=False,
                                             use_tc_tiling_on_sc=False),
        out_type=jax.ShapeDtypeStruct((NW, KP1, EPW), jnp.float32),
        scratch_types=[
            pltpu.VMEM((EPW,), jnp.int32),
            pltpu.VMEM((EPW,), jnp.int32),
            pltpu.VMEM((EPW * K,), jnp.int32),
            pltpu.VMEM((2 * C, D), jnp.float32),
            pltpu.VMEM((2 * C, D), jnp.float32),
            pltpu.VMEM((2 * C * K, D), jnp.float32),
            pltpu.VMEM((KP1, EPW), jnp.float32),
            pltpu.SemaphoreType.DMA,
            pltpu.SemaphoreType.DMA,
        ],
    )
    def sc_scores(emb_hbm, ctx_hbm, tgt_hbm, ctxi_hbm, neg_hbm, out_hbm,
                  tgt_v, ctxi_v, neg_v, vw_v, vc_v, vn_v, sc_v, semA, semB):
        wid = lax.axis_index("s") * NC + lax.axis_index("c")
        base = wid * EPW
        iota = lax.iota(jnp.int32, LANES)
        pltpu.sync_copy(tgt_hbm.at[pl.ds(base, EPW)], tgt_v)
        pltpu.sync_copy(ctxi_hbm.at[pl.ds(base, EPW)], ctxi_v)
        pltpu.sync_copy(neg_hbm.at[pl.ds(base * K, EPW * K)], neg_v)

        def copies(c, p, sem):
            s0 = pl.multiple_of(c * C, C)
            n0 = pl.multiple_of(c * C * K, C * K)
            cs = [
                pltpu.make_async_copy(
                    emb_hbm.at[tgt_v.at[pl.ds(s0, C)]],
                    vw_v.at[pl.ds(p * C, C)], sem),
                pltpu.make_async_copy(
                    ctx_hbm.at[ctxi_v.at[pl.ds(s0, C)]],
                    vc_v.at[pl.ds(p * C, C)], sem),
            ]
            for s in range(NSPLIT):
                cs.append(pltpu.make_async_copy(
                    ctx_hbm.at[neg_v.at[pl.ds(n0 + s * SEG, SEG)]],
                    vn_v.at[pl.ds(p * C * K + s * SEG, SEG)], sem))
            return cs

        def fire(c, p, sem):
            for cp in copies(c, p, sem):
                cp.start()

        def drain(c, p, sem):
            for cp in copies(c, p, sem):
                cp.wait()

        def compute(c, p):
            for g in range(NG):
                rows = p * C + g * LANES + iota
                nbase = p * C * K + (g * LANES + iota) * K

                def body(d, accs):
                    dcol = jnp.full((LANES,), d, jnp.int32)
                    vw_d = plsc.load_gather(vw_v, [rows, dcol])
                    vc_d = plsc.load_gather(vc_v, [rows, dcol])
                    new = [accs[0] + vw_d * vc_d]
                    for k in range(K):
                        vn_d = plsc.load_gather(vn_v, [nbase + k, dcol])
                        new.append(accs[k + 1] + vn_d * vw_d)
                    return tuple(new)

                accs = lax.fori_loop(
                    0, D, body,
                    tuple(jnp.zeros((LANES,), jnp.float32) for _ in range(KP1)))
                off = c * C + g * LANES
                for k in range(KP1):
                    sc_v[k, pl.ds(off, LANES)] = accs[k]

        fire(0, 0, semA)
        fire(1, 1, semB)

        def pair_body(j, carry):
            c0 = 2 * j
            drain(c0, 0, semA)
            compute(c0, 0)
            fire(c0 + 2, 0, semA)
            c1 = c0 + 1
            drain(c1, 1, semB)
            compute(c1, 1)
            fire(c1 + 2, 1, semB)
            return carry

        lax.fori_loop(0, NCHUNK // 2 - 1, pair_body, 0)
        cl = NCHUNK - 2
        drain(cl, 0, semA)
        compute(cl, 0)
        drain(cl + 1, 1, semB)
        compute(cl + 1, 1)
        pltpu.sync_copy(sc_v, out_hbm.at[wid])

    return sc_scores


@functools.lru_cache(maxsize=None)
def _make_loss(B, K):
    KP1 = K + 1
    EPW = B // NW

    def loss_body(s_ref, o_ref):
        x = s_ref[...]
        r = lax.broadcasted_iota(jnp.int32, x.shape, 0)
        t = jnp.where(r % KP1 == 0, -x, x)
        sp = jnp.maximum(t, 0.0) + jnp.log(1.0 + jnp.exp(-jnp.abs(t)))
        o_ref[0, 0] = jnp.sum(sp) / B

    def loss(scores):
        out = pl.pallas_call(
            loss_body,
            out_shape=jax.ShapeDtypeStruct((1, 1), jnp.float32),
            out_specs=pl.BlockSpec(memory_space=pltpu.SMEM),
        )(scores.reshape(NW * KP1, EPW))
        return out[0, 0]

    return loss


def kernel(target, context, negatives, emb, ctx_emb):
    B, = target.shape
    _, K = negatives.shape
    _, D = emb.shape
    tgt = target.astype(jnp.int32)
    ctxi = context.astype(jnp.int32)
    neg = negatives.astype(jnp.int32).reshape(B * K)
    scores = _make_sc_scores(B, K, D)(emb, ctx_emb, tgt, ctxi, neg)
    return _make_loss(B, K)(scores)
